# baseline (speedup 1.0000x reference)
import functools

import jax
import jax.numpy as jnp
from jax import lax
from jax.experimental import pallas as pl
from jax.experimental.pallas import tpu as pltpu
from jax.experimental.pallas import tpu_sc as plsc

D = 128
B = 1024
S1 = 25
S2 = 10
H = 128

NC, NS = 2, 16
NW = NC * NS

NB_W = B // NW
ROW_I = S1 * S2
SLAB_H = NB_W // 2
G0_W = B // NW
G1_W = (B * S1) // NW
SEG_W = (B * S1) // NW

CH_SEG = 32
CH_ROWS = CH_SEG * S2
N_CH = SEG_W // CH_SEG
N_PAIRS = N_CH // 2
G1_Q = G1_W // 4


def _tree_sum(vals):
  while len(vals) > 1:
    nxt = [vals[i] + vals[i + 1] for i in range(0, len(vals) - 1, 2)]
    if len(vals) % 2:
      nxt.append(vals[-1])
    vals = nxt
  return vals[0]


def _sc_gather(features, batch_nodes, n1_flat, n2_rows):
  mesh = plsc.VectorSubcoreMesh(core_axis_name="c", subcore_axis_name="s")

  @functools.partial(
      pl.kernel,
      out_type=(
          jax.ShapeDtypeStruct((B * S1, D), jnp.float32),
          jax.ShapeDtypeStruct((B * S1, D), jnp.float32),
          jax.ShapeDtypeStruct((B, D), jnp.float32),
      ),
      mesh=mesh,
      scratch_types=[
          pltpu.VMEM((SLAB_H, ROW_I), jnp.int32),
          pltpu.VMEM((SEG_W * S2,), jnp.int32),
          pltpu.VMEM((G1_W,), jnp.int32),
          pltpu.VMEM((G0_W,), jnp.int32),
          pltpu.VMEM((2, CH_ROWS, D), jnp.float32),
          pltpu.VMEM((2, CH_SEG, D), jnp.float32),
          pltpu.VMEM((G1_Q, D), jnp.float32),
          pltpu.SemaphoreType.DMA,
          pltpu.SemaphoreType.DMA,
          pltpu.SemaphoreType.DMA,
          pltpu.SemaphoreType.DMA,
          pltpu.SemaphoreType.DMA,
          pltpu.SemaphoreType.DMA,
      ],
  )
  def k(feat_hbm, idx0_hbm, idx1_hbm, idx2_hbm, g1_hbm, m2_hbm, g0_hbm,
        idx2s_v, idx2_v, idx1_v, idx0_v, gbuf, obuf, g1buf,
        gsem0, gsem1, osem0, osem1, g1sem, g0sem):
    wid = lax.axis_index("s") * NC + lax.axis_index("c")

    pltpu.sync_copy(
        idx1_hbm.at[pl.ds(pl.multiple_of(wid * G1_W, 8), G1_W)], idx1_v)
    pltpu.sync_copy(
        idx0_hbm.at[pl.ds(pl.multiple_of(wid * G0_W, 8), G0_W)], idx0_v)

    gsems = (gsem0, gsem1)
    osems = (osem0, osem1)
    inv = jnp.float32(1.0 / S2)

    def g1_quarter(q):
      src = idx1_v.at[pl.ds(pl.multiple_of(q * G1_Q, 8), G1_Q)]
      return feat_hbm.at[src], g1buf, g1sem

    pltpu.async_copy(feat_hbm.at[idx0_v], obuf.at[0], g0sem)
    s, d, sm = g1_quarter(0)
    pltpu.async_copy(s, d, sm)

    for half in range(2):
      pltpu.sync_copy(
          idx2_hbm.at[pl.ds(
              pl.multiple_of(wid * NB_W + half * SLAB_H, 8), SLAB_H)],
          idx2s_v)

      def flat_body(bb, carry, half=half):
        base = (half * SLAB_H + bb) * ROW_I
        for s16 in range(ROW_I // 16):
          sl = pl.ds(s16 * 16, 16)
          idx2_v[pl.ds(base + s16 * 16, 16)] = idx2s_v[bb, sl]
        tail = ROW_I - 16
        idx2_v[pl.ds(base + tail, 16)] = idx2s_v[bb, pl.ds(tail, 16)]
        return carry
      lax.fori_loop(0, SLAB_H, flat_body, 0)

    pltpu.make_async_copy(feat_hbm.at[idx0_v], obuf.at[0], g0sem).wait()
    pltpu.sync_copy(obuf.at[0],
                    g0_hbm.at[pl.ds(pl.multiple_of(wid * G0_W, 8), G0_W)])

    def gather_src(g, b):
      off = pl.multiple_of(g * CH_ROWS, 8)
      return (feat_hbm.at[idx2_v.at[pl.ds(off, CH_ROWS)]], gbuf.at[b],
              gsems[b])

    def fire(g, b):
      src, dst, sem = gather_src(g, b)
      pltpu.async_copy(src, dst, sem)

    def wait_gather(g, b):
      src, dst, sem = gather_src(g, b)
      pltpu.make_async_copy(src, dst, sem).wait()

    def mean_out(g, b):
      dst_off = pl.multiple_of(wid * SEG_W + g * CH_SEG, 8)
      return obuf.at[b], m2_hbm.at[pl.ds(dst_off, CH_SEG)], osems[b]

    def reduce_chunk(g, b):
      def seg_body(ci, c2):
        base = ci * S2
        for j in range(D // 16):
          sl = pl.ds(j * 16, 16)
          acc = _tree_sum([gbuf[b, base + r, sl] for r in range(S2)])
          obuf[b, ci, sl] = acc * inv
        return c2
      lax.fori_loop(0, CH_SEG, seg_body, 0)

    fire(0, 0)
    fire(1, 1)

    def pair_body(p, carry):
      for b in range(2):
        g = p * 2 + b
        wait_gather(g, b)

        @pl.when(p >= 1)
        def _(g=g, b=b):
          src, dst, sem = mean_out(g - 2, b)
          pltpu.make_async_copy(src, dst, sem).wait()

        reduce_chunk(g, b)

        @pl.when(g + 2 < N_CH)
        def _(g=g, b=b):
          fire(g + 2, b)

        src, dst, sem = mean_out(g, b)
        pltpu.async_copy(src, dst, sem)
      return carry

    for q in range(4):
      if q:
        lax.fori_loop(q * N_PAIRS // 4, (q + 1) * N_PAIRS // 4, pair_body, 0)
      else:
        lax.fori_loop(0, N_PAIRS // 4, pair_body, 0)
      s, d, sm = g1_quarter(q)
      pltpu.make_async_copy(s, d, sm).wait()
      pltpu.sync_copy(
          g1buf,
          g1_hbm.at[pl.ds(pl.multiple_of(wid * G1_W + q * G1_Q, 8), G1_Q)])
      if q < 3:
        s, d, sm = g1_quarter(q + 1)
        pltpu.async_copy(s, d, sm)

    g_last = N_CH - 1
    wait_gather(g_last, g_last % 2)
    src, dst, sem = mean_out(g_last - 2, g_last % 2)
    pltpu.make_async_copy(src, dst, sem).wait()
    reduce_chunk(g_last, g_last % 2)
    src, dst, sem = mean_out(g_last, g_last % 2)
    pltpu.async_copy(src, dst, sem)
    for g in (N_CH - 2, N_CH - 1):
      src, dst, sem = mean_out(g, g % 2)
      pltpu.make_async_copy(src, dst, sem).wait()

  return k(features, batch_nodes, n1_flat, n2_rows)


TC_G = 128
TC_R = TC_G * S1
TC_STEPS = B // TC_G


def _tc_dense(g1v, m2v, g0, w_s1, w_n1, w_s2, w_n2, smat):
  def body(g1_ref, m2_ref, g0_ref, ws1, wn1, ws2, wn2, s_ref, out_ref,
           acc_a, acc_b, acc_m):
    t = pl.program_id(0)
    x = g1_ref[0]
    m = m2_ref[0]
    smx = s_ref[...]
    ya = jnp.maximum(
        jnp.dot(x, ws1[...], preferred_element_type=jnp.float32), 0.0)
    yb = jnp.maximum(
        jnp.dot(m, wn1[...], preferred_element_type=jnp.float32), 0.0)
    rows = pl.ds(t * TC_G, TC_G)
    acc_a[rows, :] = jnp.dot(smx, ya, preferred_element_type=jnp.float32)
    acc_b[rows, :] = jnp.dot(smx, yb, preferred_element_type=jnp.float32)
    acc_m[rows, :] = jnp.dot(smx, x, preferred_element_type=jnp.float32)

    @pl.when(t == TC_STEPS - 1)
    def _():
      inv = jnp.float32(1.0 / S1)
      h1s = jnp.maximum(
          jnp.dot(g0_ref[...], ws1[...], preferred_element_type=jnp.float32),
          0.0)
      h1n = jnp.maximum(
          jnp.dot(acc_m[...] * inv, wn1[...],
                  preferred_element_type=jnp.float32), 0.0)
      w2 = ws2[...]
      self2 = (jnp.dot(h1s, w2[:H], preferred_element_type=jnp.float32)
               + jnp.dot(h1n, w2[H:], preferred_element_type=jnp.float32))
      wn = wn2[...]
      n2 = (jnp.dot(acc_a[...] * inv, wn[:H],
                    preferred_element_type=jnp.float32)
            + jnp.dot(acc_b[...] * inv, wn[H:],
                      preferred_element_type=jnp.float32))
      h2 = jnp.maximum(jnp.concatenate([self2, n2], axis=1), 0.0)
      nrm = jnp.sqrt(jnp.sum(h2 * h2, axis=1, keepdims=True)) + 1e-12
      out_ref[...] = h2 / nrm

  return pl.pallas_call(
      body,
      grid=(TC_STEPS,),
      in_specs=[
          pl.BlockSpec((1, TC_R, D), lambda t: (t, 0, 0)),
          pl.BlockSpec((1, TC_R, D), lambda t: (t, 0, 0)),
          pl.BlockSpec((B, D), lambda t: (0, 0)),
          pl.BlockSpec((D, H), lambda t: (0, 0)),
          pl.BlockSpec((D, H), lambda t: (0, 0)),
          pl.BlockSpec((2 * H, H), lambda t: (0, 0)),
          pl.BlockSpec((2 * H, H), lambda t: (0, 0)),
          pl.BlockSpec((TC_G, TC_R), lambda t: (0, 0)),
      ],
      out_specs=pl.BlockSpec((B, 2 * H), lambda t: (0, 0)),
      out_shape=jax.ShapeDtypeStruct((B, 2 * H), jnp.float32),
      scratch_shapes=[
          pltpu.VMEM((B, H), jnp.float32),
          pltpu.VMEM((B, H), jnp.float32),
          pltpu.VMEM((B, D), jnp.float32),
      ],
  )(g1v, m2v, g0, w_s1, w_n1, w_s2, w_n2, smat)


def kernel(features, batch_nodes, neigh1, neigh2,
           W_self1, W_neigh1, W_self2, W_neigh2):
  g1, m2, g0 = _sc_gather(features, batch_nodes,
                          neigh1.reshape(-1), neigh2.reshape(B, ROW_I))
  smat = (jnp.arange(TC_G, dtype=jnp.int32)[:, None]
          == (jnp.arange(TC_R, dtype=jnp.int32)[None, :] // S1)
          ).astype(jnp.float32)
  return _tc_dense(g1.reshape(TC_STEPS, TC_R, D),
                   m2.reshape(TC_STEPS, TC_R, D),
                   g0, W_self1, W_neigh1, W_self2, W_neigh2, smat)


# speedup vs baseline: 1.0255x; 1.0255x over previous
import functools

import jax
import jax.numpy as jnp
from jax import lax
from jax.experimental import pallas as pl
from jax.experimental.pallas import tpu as pltpu
from jax.experimental.pallas import tpu_sc as plsc

D = 128
B = 1024
S1 = 25
S2 = 10
H = 128

NC, NS = 2, 16
NW = NC * NS

NB_W = B // NW
ROW_I = S1 * S2
G0_W = B // NW
G1_W = (B * S1) // NW
SEG_W = (B * S1) // NW

CH_SEG = 16
CH_ROWS = CH_SEG * S2
N_CH = SEG_W // CH_SEG
G1_Q = G1_W // 4


def _tree_sum(vals):
  while len(vals) > 1:
    nxt = [vals[i] + vals[i + 1] for i in range(0, len(vals) - 1, 2)]
    if len(vals) % 2:
      nxt.append(vals[-1])
    vals = nxt
  return vals[0]


def _sc_gather(features, batch_nodes, n1_flat, n2_rows):
  mesh = plsc.VectorSubcoreMesh(core_axis_name="c", subcore_axis_name="s")

  @functools.partial(
      pl.kernel,
      out_type=(
          jax.ShapeDtypeStruct((B * S1, D), jnp.float32),
          jax.ShapeDtypeStruct((B * S1, D), jnp.float32),
          jax.ShapeDtypeStruct((B, D), jnp.float32),
      ),
      mesh=mesh,
      scratch_types=[
          pltpu.VMEM((NB_W, ROW_I), jnp.int32),
          pltpu.VMEM((SEG_W * S2,), jnp.int32),
          pltpu.VMEM((G1_W,), jnp.int32),
          pltpu.VMEM((G0_W,), jnp.int32),
          pltpu.VMEM((2, CH_ROWS, D), jnp.float32),
          pltpu.VMEM((2, CH_SEG, D), jnp.float32),
          pltpu.VMEM((G1_Q, D), jnp.float32),
          pltpu.VMEM((G0_W, D), jnp.float32),
          pltpu.SemaphoreType.DMA,
          pltpu.SemaphoreType.DMA,
          pltpu.SemaphoreType.DMA,
          pltpu.SemaphoreType.DMA,
          pltpu.SemaphoreType.DMA,
          pltpu.SemaphoreType.DMA,
      ],
  )
  def k(feat_hbm, idx0_hbm, idx1_hbm, idx2_hbm, g1_hbm, m2_hbm, g0_hbm,
        idx2s_v, idx2_v, idx1_v, idx0_v, gbuf, obuf, g1buf, g0buf,
        gsem0, gsem1, osem0, osem1, g1sem, g0sem):
    wid = lax.axis_index("s") * NC + lax.axis_index("c")

    pltpu.sync_copy(
        idx1_hbm.at[pl.ds(pl.multiple_of(wid * G1_W, 8), G1_W)], idx1_v)
    pltpu.sync_copy(
        idx0_hbm.at[pl.ds(pl.multiple_of(wid * G0_W, 8), G0_W)], idx0_v)

    gsems = (gsem0, gsem1)
    osems = (osem0, osem1)
    inv = jnp.float32(1.0 / S2)

    def g1_quarter(q):
      src = idx1_v.at[pl.ds(pl.multiple_of(q * G1_Q, 8), G1_Q)]
      return feat_hbm.at[src], g1buf, g1sem

    pltpu.async_copy(feat_hbm.at[idx0_v], g0buf, g0sem)
    s, d, sm = g1_quarter(0)
    pltpu.async_copy(s, d, sm)

    pltpu.sync_copy(
        idx2_hbm.at[pl.ds(pl.multiple_of(wid * NB_W, 8), NB_W)], idx2s_v)

    def flat_body(bb, carry):
      base = bb * ROW_I
      for s16 in range(ROW_I // 16):
        sl = pl.ds(s16 * 16, 16)
        idx2_v[pl.ds(base + s16 * 16, 16)] = idx2s_v[bb, sl]
      tail = ROW_I - 16
      idx2_v[pl.ds(base + tail, 16)] = idx2s_v[bb, pl.ds(tail, 16)]
      return carry
    lax.fori_loop(0, NB_W, flat_body, 0)

    def gather_src(g, b):
      off = pl.multiple_of(g * CH_ROWS, 8)
      return (feat_hbm.at[idx2_v.at[pl.ds(off, CH_ROWS)]], gbuf.at[b],
              gsems[b])

    def fire(g, b):
      src, dst, sem = gather_src(g, b)
      pltpu.async_copy(src, dst, sem)

    def wait_gather(g, b):
      src, dst, sem = gather_src(g, b)
      pltpu.make_async_copy(src, dst, sem).wait()

    def mean_out(g, b):
      dst_off = pl.multiple_of(wid * SEG_W + g * CH_SEG, 8)
      return obuf.at[b], m2_hbm.at[pl.ds(dst_off, CH_SEG)], osems[b]

    fire(0, 0)
    fire(1, 1)

    def pair_body(p, carry):
      for b in range(2):
        g = p * 2 + b
        wait_gather(g, b)

        @pl.when(p >= 1)
        def _(g=g, b=b):
          src, dst, sem = mean_out(g - 2, b)
          pltpu.make_async_copy(src, dst, sem).wait()

        def seg_body(ci, c2, b=b):
          base = ci * S2
          for j in range(D // 16):
            sl = pl.ds(j * 16, 16)
            acc = _tree_sum([gbuf[b, base + r, sl] for r in range(S2)])
            obuf[b, ci, sl] = acc * inv
          return c2
        lax.fori_loop(0, CH_SEG, seg_body, 0)

        @pl.when(g + 2 < N_CH)
        def _(g=g, b=b):
          fire(g + 2, b)

        src, dst, sem = mean_out(g, b)
        pltpu.async_copy(src, dst, sem)
      return carry

    n_pairs = N_CH // 2
    lax.fori_loop(0, n_pairs // 4, pair_body, 0)

    pltpu.make_async_copy(feat_hbm.at[idx0_v], g0buf, g0sem).wait()
    pltpu.sync_copy(g0buf,
                    g0_hbm.at[pl.ds(pl.multiple_of(wid * G0_W, 8), G0_W)])

    for q in range(4):
      if q:
        lax.fori_loop(q * n_pairs // 4, (q + 1) * n_pairs // 4, pair_body, 0)
      s, d, sm = g1_quarter(q)
      pltpu.make_async_copy(s, d, sm).wait()
      pltpu.sync_copy(
          g1buf,
          g1_hbm.at[pl.ds(pl.multiple_of(wid * G1_W + q * G1_Q, 8), G1_Q)])
      if q < 3:
        s, d, sm = g1_quarter(q + 1)
        pltpu.async_copy(s, d, sm)

    for b in range(2):
      src, dst, sem = mean_out(N_CH - 2 + b, b)
      pltpu.make_async_copy(src, dst, sem).wait()

  return k(features, batch_nodes, n1_flat, n2_rows)


TC_G = 128
TC_R = TC_G * S1
TC_STEPS = B // TC_G


def _tc_dense(g1v, m2v, g0, w_s1, w_n1, w_s2, w_n2, smat):
  def body(g1_ref, m2_ref, g0_ref, ws1, wn1, ws2, wn2, s_ref, out_ref,
           acc_a, acc_b, acc_m):
    t = pl.program_id(0)
    x = g1_ref[0]
    m = m2_ref[0]
    smx = s_ref[...]
    ya = jnp.maximum(
        jnp.dot(x, ws1[...], preferred_element_type=jnp.float32), 0.0)
    yb = jnp.maximum(
        jnp.dot(m, wn1[...], preferred_element_type=jnp.float32), 0.0)
    rows = pl.ds(t * TC_G, TC_G)
    acc_a[rows, :] = jnp.dot(smx, ya, preferred_element_type=jnp.float32)
    acc_b[rows, :] = jnp.dot(smx, yb, preferred_element_type=jnp.float32)
    acc_m[rows, :] = jnp.dot(smx, x, preferred_element_type=jnp.float32)

    @pl.when(t == TC_STEPS - 1)
    def _():
      inv = jnp.float32(1.0 / S1)
      h1s = jnp.maximum(
          jnp.dot(g0_ref[...], ws1[...], preferred_element_type=jnp.float32),
          0.0)
      h1n = jnp.maximum(
          jnp.dot(acc_m[...] * inv, wn1[...],
                  preferred_element_type=jnp.float32), 0.0)
      w2 = ws2[...]
      self2 = (jnp.dot(h1s, w2[:H], preferred_element_type=jnp.float32)
               + jnp.dot(h1n, w2[H:], preferred_element_type=jnp.float32))
      wn = wn2[...]
      n2 = (jnp.dot(acc_a[...] * inv, wn[:H],
                    preferred_element_type=jnp.float32)
            + jnp.dot(acc_b[...] * inv, wn[H:],
                      preferred_element_type=jnp.float32))
      h2 = jnp.maximum(jnp.concatenate([self2, n2], axis=1), 0.0)
      nrm = jnp.sqrt(jnp.sum(h2 * h2, axis=1, keepdims=True)) + 1e-12
      out_ref[...] = h2 / nrm

  return pl.pallas_call(
      body,
      grid=(TC_STEPS,),
      in_specs=[
          pl.BlockSpec((1, TC_R, D), lambda t: (t, 0, 0)),
          pl.BlockSpec((1, TC_R, D), lambda t: (t, 0, 0)),
          pl.BlockSpec((B, D), lambda t: (0, 0)),
          pl.BlockSpec((D, H), lambda t: (0, 0)),
          pl.BlockSpec((D, H), lambda t: (0, 0)),
          pl.BlockSpec((2 * H, H), lambda t: (0, 0)),
          pl.BlockSpec((2 * H, H), lambda t: (0, 0)),
          pl.BlockSpec((TC_G, TC_R), lambda t: (0, 0)),
      ],
      out_specs=pl.BlockSpec((B, 2 * H), lambda t: (0, 0)),
      out_shape=jax.ShapeDtypeStruct((B, 2 * H), jnp.float32),
      scratch_shapes=[
          pltpu.VMEM((B, H), jnp.float32),
          pltpu.VMEM((B, H), jnp.float32),
          pltpu.VMEM((B, D), jnp.float32),
      ],
  )(g1v, m2v, g0, w_s1, w_n1, w_s2, w_n2, smat)


def kernel(features, batch_nodes, neigh1, neigh2,
           W_self1, W_neigh1, W_self2, W_neigh2):
  g1, m2, g0 = _sc_gather(features, batch_nodes,
                          neigh1.reshape(-1), neigh2.reshape(B, ROW_I))
  smat = (jnp.arange(TC_G, dtype=jnp.int32)[:, None]
          == (jnp.arange(TC_R, dtype=jnp.int32)[None, :] // S1)
          ).astype(jnp.float32)
  return _tc_dense(g1.reshape(TC_STEPS, TC_R, D),
                   m2.reshape(TC_STEPS, TC_R, D),
                   g0, W_self1, W_neigh1, W_self2, W_neigh2, smat)
